# baseline (speedup 1.0000x reference)
import functools

import jax
import jax.numpy as jnp
from jax import lax
from jax.experimental import pallas as pl
from jax.experimental.pallas import tpu as pltpu
from jax.experimental.pallas import tpu_sc as plsc

N = 10000
E = 160000
D = 128
K = 16

_info = plsc.get_sparse_core_info()
_NC = _info.num_cores
_NS = _info.num_subcores
_NW = _NC * _NS
_LANES = 8


def _mesh():
    return plsc.VectorSubcoreMesh(core_axis_name="c", subcore_axis_name="s")


@functools.partial(jax.jit, static_argnames=("rows",))
def _sc_gathersum(table, idx_flat, *, rows):
    CH = 8
    nch = rows // CH
    q, rem = divmod(nch, _NW)

    @functools.partial(
        pl.kernel,
        out_type=jax.ShapeDtypeStruct((rows, D), jnp.float32),
        mesh=_mesh(),
        scratch_types=[
            pltpu.VMEM((CH * K,), jnp.int32),
            pltpu.VMEM((CH * K, D), jnp.float32),
            pltpu.VMEM((CH, D), jnp.float32),
            pltpu.SemaphoreType.DMA,
        ],
    )
    def k(table_ref, idx_ref, out_ref, idx_v, rows_v, out_v, sem):
        wid = lax.axis_index("s") * _NC + lax.axis_index("c")
        cnt = q + jnp.where(wid < rem, 1, 0)

        @pl.loop(0, cnt)
        def _(i):
            ch = wid + i * _NW
            pltpu.sync_copy(idx_ref.at[pl.ds(ch * CH * K, CH * K)], idx_v)
            pltpu.async_copy(table_ref.at[idx_v], rows_v, sem).wait()

            @pl.loop(0, CH)
            def _(r):
                for c in range(_LANES):
                    acc = rows_v[r * K, pl.ds(c * 16, 16)]
                    for kk in range(1, K):
                        acc = acc + rows_v[r * K + kk, pl.ds(c * 16, 16)]
                    out_v[r, pl.ds(c * 16, 16)] = acc

            pltpu.sync_copy(out_v, out_ref.at[pl.ds(ch * CH, CH)])

    return k(table, idx_flat)


@jax.jit
def _sc_gather_rows(table, idx):
    CH = 64
    nch = E // CH
    q, rem = divmod(nch, _NW)

    @functools.partial(
        pl.kernel,
        out_type=jax.ShapeDtypeStruct((E, D), jnp.float32),
        mesh=_mesh(),
        scratch_types=[
            pltpu.VMEM((CH,), jnp.int32),
            pltpu.VMEM((CH, D), jnp.float32),
            pltpu.SemaphoreType.DMA,
        ],
    )
    def k(table_ref, idx_ref, out_ref, idx_v, rows_v, sem):
        wid = lax.axis_index("s") * _NC + lax.axis_index("c")
        cnt = q + jnp.where(wid < rem, 1, 0)

        @pl.loop(0, cnt)
        def _(i):
            ch = wid + i * _NW
            pltpu.sync_copy(idx_ref.at[pl.ds(ch * CH, CH)], idx_v)
            pltpu.async_copy(table_ref.at[idx_v], rows_v, sem).wait()
            pltpu.sync_copy(rows_v, out_ref.at[pl.ds(ch * CH, CH)])

    return k(table, idx)


@jax.jit
def _sc_scatter_add(vals, dst):
    CH = 16
    nch = E // CH
    q, rem = divmod(nch, _NW)
    RB = 16
    nrch = N // RB
    rq, rrem = divmod(nrch, _NS)

    @functools.partial(
        pl.kernel,
        out_type=jax.ShapeDtypeStruct((_NC, N, D), jnp.float32),
        mesh=_mesh(),
        scratch_types=[
            pltpu.VMEM((CH,), jnp.int32),
            pltpu.VMEM((CH, D), jnp.float32),
            pltpu.VMEM((RB, D), jnp.float32),
            pltpu.VMEM((RB, D), jnp.float32),
            pltpu.VMEM_SHARED((N, D), jnp.float32),
            pltpu.SemaphoreType.DMA,
        ],
    )
    def k(vals_ref, dst_ref, out_ref, idx_v, rows_v, zbuf, obuf, acc, sem):
        cid = lax.axis_index("c")
        sid = lax.axis_index("s")
        wid = sid * _NC + cid
        rcnt = rq + jnp.where(sid < rrem, 1, 0)

        for r in range(RB):
            for c in range(_LANES):
                zbuf[r, pl.ds(c * 16, 16)] = jnp.zeros((16,), jnp.float32)

        @pl.loop(0, rcnt)
        def _(j):
            pltpu.sync_copy(zbuf, acc.at[pl.ds((sid + j * _NS) * RB, RB)])

        plsc.subcore_barrier()

        cnt = q + jnp.where(wid < rem, 1, 0)

        @pl.loop(0, cnt)
        def _(i):
            ch = wid + i * _NW
            pltpu.sync_copy(dst_ref.at[pl.ds(ch * CH, CH)], idx_v)
            pltpu.sync_copy(vals_ref.at[pl.ds(ch * CH, CH)], rows_v)
            pltpu.sync_copy(rows_v, acc.at[idx_v], add=True)

        plsc.subcore_barrier()

        @pl.loop(0, rcnt)
        def _(j):
            off = (sid + j * _NS) * RB
            pltpu.sync_copy(acc.at[pl.ds(off, RB)], obuf)
            pltpu.sync_copy(obuf, out_ref.at[cid, pl.ds(off, RB)])

    return k(vals, dst)


def _relu_half(h):
    col = lax.broadcasted_iota(jnp.int32, h.shape, 1)
    return jnp.where(col >= D // 2, jnp.maximum(h, 0.0), h)


def _pre_body(refs, h_ref, stats_ref):
    (base_ref, deg_ref, sa_ref, sb_ref, o1_ref, o2_ref,
     wm, wd, wa, wb, wo, bias_ref) = refs
    b = base_ref[...]
    h = jnp.dot(b, wm[...], preferred_element_type=jnp.float32)
    h = h + jnp.dot(b * deg_ref[...], wd[...], preferred_element_type=jnp.float32)
    h = h + jnp.dot(sa_ref[...], wa[...], preferred_element_type=jnp.float32)
    h = h + jnp.dot(sb_ref[...], wb[...], preferred_element_type=jnp.float32)
    other = o1_ref[...] if o2_ref is None else o1_ref[...] + o2_ref[...]
    h = h + jnp.dot(other, wo[...], preferred_element_type=jnp.float32)
    h = h + bias_ref[...]
    h = _relu_half(h)
    h_ref[...] = h

    @pl.when(pl.program_id(0) == 0)
    def _():
        stats_ref[...] = jnp.zeros_like(stats_ref)

    stats_ref[0:1, :] = stats_ref[0:1, :] + jnp.sum(h, axis=0, keepdims=True)
    stats_ref[1:2, :] = stats_ref[1:2, :] + jnp.sum(h * h, axis=0, keepdims=True)


def _tc_pre(base, deg, sa, sb, others, wm, wd, wa, wb, wo, bias, *, blk):
    rows = base.shape[0]
    grid = rows // blk
    two = len(others) == 2

    def body(base_ref, deg_ref, sa_ref, sb_ref, o1_ref, *rest):
        if two:
            o2_ref, wm_r, wd_r, wa_r, wb_r, wo_r, bias_ref, h_ref, stats_ref = rest
        else:
            wm_r, wd_r, wa_r, wb_r, wo_r, bias_ref, h_ref, stats_ref = rest
            o2_ref = None
        _pre_body((base_ref, deg_ref, sa_ref, sb_ref, o1_ref, o2_ref,
                   wm_r, wd_r, wa_r, wb_r, wo_r, bias_ref), h_ref, stats_ref)

    row_spec = pl.BlockSpec((blk, D), lambda i: (i, 0))
    full_spec = pl.BlockSpec((D, D), lambda i: (0, 0))
    vec_spec = pl.BlockSpec((1, D), lambda i: (0, 0))
    in_specs = [row_spec, pl.BlockSpec((blk, 1), lambda i: (i, 0)),
                row_spec, row_spec, row_spec]
    if two:
        in_specs.append(row_spec)
    in_specs += [full_spec] * 5 + [vec_spec]

    return pl.pallas_call(
        body,
        grid=(grid,),
        in_specs=in_specs,
        out_specs=[row_spec, pl.BlockSpec((8, D), lambda i: (0, 0))],
        out_shape=[jax.ShapeDtypeStruct((rows, D), jnp.float32),
                   jax.ShapeDtypeStruct((8, D), jnp.float32)],
    )(base, deg, sa, sb, *others, wm, wd, wa, wb, wo, bias)


def _tc_bn(h, stats, s, b, *, blk):
    rows = h.shape[0]
    grid = rows // blk
    inv_n = 1.0 / rows

    def body(h_ref, stats_ref, s_ref, b_ref, o_ref):
        m = stats_ref[0:1, :] * inv_n
        v = stats_ref[1:2, :] * inv_n - m * m
        scale = lax.rsqrt(v + 1e-5) * s_ref[...]
        o_ref[...] = (h_ref[...] - m) * scale + b_ref[...]

    row_spec = pl.BlockSpec((blk, D), lambda i: (i, 0))
    vec_spec = pl.BlockSpec((1, D), lambda i: (0, 0))
    return pl.pallas_call(
        body,
        grid=(grid,),
        in_specs=[row_spec, pl.BlockSpec((8, D), lambda i: (0, 0)),
                  vec_spec, vec_spec],
        out_specs=row_spec,
        out_shape=jax.ShapeDtypeStruct((rows, D), jnp.float32),
    )(h, stats, s.reshape(1, D), b.reshape(1, D))


def kernel(x, y, deg_g, deg_lg, pm_pd, g_t, g_tt, lg_t, lg_tt, edge_dst,
           W_tx, b_tx, W_td, b_td, W_ty, b_ty, W_t0, b_t0, W_t1, b_t1,
           W_gy, b_gy, W_gd, b_gd, W_gx, b_gx, W_g0, b_g0, W_g1, b_g1,
           bnx_s, bnx_b, bny_s, bny_b):
    sxt = _sc_gathersum(x, g_t.reshape(-1), rows=N)
    sxtt = _sc_gathersum(x, g_tt.reshape(-1), rows=N)
    syt = _sc_gathersum(y, lg_t.reshape(-1), rows=E)
    sytt = _sc_gathersum(y, lg_tt.reshape(-1), rows=E)
    px = _sc_gather_rows(x, pm_pd)
    py = _sc_scatter_add(y, edge_dst)

    bias_x = (b_tx + b_td + b_t0 + b_t1 + b_ty).reshape(1, D)
    bias_y = (b_gy + b_gd + b_g0 + b_g1 + b_gx).reshape(1, D)

    hx, stx = _tc_pre(x, deg_g, sxt, sxtt, [py[0], py[1]],
                      W_tx.T, W_td.T, W_t0.T, W_t1.T, W_ty.T, bias_x,
                      blk=2000)
    hy, sty = _tc_pre(y, deg_lg, syt, sytt, [px],
                      W_gy.T, W_gd.T, W_g0.T, W_g1.T, W_gx.T, bias_y,
                      blk=2000)

    xn = _tc_bn(hx, stx, bnx_s, bnx_b, blk=2000)
    yn = _tc_bn(hy, sty, bny_s, bny_b, blk=2000)
    return (xn, yn)


# speedup vs baseline: 1.4781x; 1.4781x over previous
import functools

import jax
import jax.numpy as jnp
from jax import lax
from jax.experimental import pallas as pl
from jax.experimental.pallas import tpu as pltpu
from jax.experimental.pallas import tpu_sc as plsc

N = 10000
E = 160000
D = 128
K = 16

_info = plsc.get_sparse_core_info()
_NC = _info.num_cores
_NS = _info.num_subcores
_NW = _NC * _NS
_LANES = 8


def _mesh():
    return plsc.VectorSubcoreMesh(core_axis_name="c", subcore_axis_name="s")


_CH = 8


def _gs_reduce(rows_v, out_v):
    @pl.loop(0, _CH)
    def _(r):
        for c in range(_LANES):
            acc = rows_v[r * K, pl.ds(c * 16, 16)]
            for kk in range(1, K):
                acc = acc + rows_v[r * K + kk, pl.ds(c * 16, 16)]
            out_v[r, pl.ds(c * 16, 16)] = acc


@functools.partial(jax.jit, static_argnames=("rows", "with_p"))
def _sc_gather_fused(table, idxA, idxB, ptable, pidx, *, rows, with_p):
    nch = rows // _CH
    q, rem = divmod(nch, _NW)
    out_types = [jax.ShapeDtypeStruct((rows, D), jnp.float32),
                 jax.ShapeDtypeStruct((rows, D), jnp.float32)]
    scratch = [
        pltpu.VMEM((_CH * K,), jnp.int32),
        pltpu.VMEM((_CH * K, D), jnp.float32),
        pltpu.VMEM((_CH, D), jnp.float32),
        pltpu.VMEM((_CH * K,), jnp.int32),
        pltpu.VMEM((_CH * K, D), jnp.float32),
        pltpu.VMEM((_CH, D), jnp.float32),
        pltpu.SemaphoreType.DMA,
        pltpu.SemaphoreType.DMA,
        pltpu.SemaphoreType.DMA,
        pltpu.SemaphoreType.DMA,
    ]
    if with_p:
        out_types.append(jax.ShapeDtypeStruct((rows, D), jnp.float32))
        scratch += [
            pltpu.VMEM((_CH,), jnp.int32),
            pltpu.VMEM((_CH, D), jnp.float32),
            pltpu.SemaphoreType.DMA,
        ]

    def k(tab_ref, idxA_ref, idxB_ref, *rest):
        if with_p:
            (ptab_ref, pidx_ref, outA, outB, outP,
             iA_v, rA_v, oA_v, iB_v, rB_v, oB_v,
             semA, semB, semOA, semOB, iP_v, rP_v, semP) = rest
        else:
            (outA, outB, iA_v, rA_v, oA_v, iB_v, rB_v, oB_v,
             semA, semB, semOA, semOB) = rest
        wid = lax.axis_index("s") * _NC + lax.axis_index("c")
        cnt = q + jnp.where(wid < rem, 1, 0)

        def start(ch, idx_ref, idx_v, rows_v, sem):
            pltpu.sync_copy(idx_ref.at[pl.ds(ch * idx_v.shape[0],
                                             idx_v.shape[0])], idx_v)
            pltpu.async_copy(tab_ref.at[idx_v], rows_v, sem)

        def startp(ch):
            pltpu.sync_copy(pidx_ref.at[pl.ds(ch * _CH, _CH)], iP_v)
            pltpu.async_copy(ptab_ref.at[iP_v], rP_v, semP)

        @pl.when(cnt > 0)
        def _():
            start(wid, idxA_ref, iA_v, rA_v, semA)
            start(wid, idxB_ref, iB_v, rB_v, semB)
            if with_p:
                startp(wid)

        @pl.loop(0, cnt)
        def _(i):
            ch = wid + i * _NW
            nxt = ch + _NW

            def stream(idx_ref, idx_v, rows_v, out_v, sem, sem_o, out_ref):
                pltpu.make_async_copy(tab_ref.at[idx_v], rows_v, sem).wait()

                @pl.when(i > 0)
                def _():
                    pltpu.make_async_copy(
                        out_v, out_ref.at[pl.ds((ch - _NW) * _CH, _CH)],
                        sem_o).wait()

                _gs_reduce(rows_v, out_v)
                pltpu.async_copy(out_v, out_ref.at[pl.ds(ch * _CH, _CH)],
                                 sem_o)

                @pl.when(i + 1 < cnt)
                def _():
                    start(nxt, idx_ref, idx_v, rows_v, sem)

            stream(idxA_ref, iA_v, rA_v, oA_v, semA, semOA, outA)
            stream(idxB_ref, iB_v, rB_v, oB_v, semB, semOB, outB)
            if with_p:
                pltpu.make_async_copy(ptab_ref.at[iP_v], rP_v, semP).wait()
                pltpu.sync_copy(rP_v, outP.at[pl.ds(ch * _CH, _CH)])

                @pl.when(i + 1 < cnt)
                def _():
                    startp(nxt)

        @pl.when(cnt > 0)
        def _():
            last = wid + (cnt - 1) * _NW
            pltpu.make_async_copy(
                oA_v, outA.at[pl.ds(last * _CH, _CH)], semOA).wait()
            pltpu.make_async_copy(
                oB_v, outB.at[pl.ds(last * _CH, _CH)], semOB).wait()

    built = pl.kernel(k, out_type=out_types, mesh=_mesh(),
                      scratch_types=scratch)
    if with_p:
        return built(table, idxA, idxB, ptable, pidx)
    return built(table, idxA, idxB)


@jax.jit
def _sc_scatter_add(vals, dst):
    CH = 16
    nch = E // CH
    q, rem = divmod(nch, _NW)
    RB = 16
    nrch = N // RB
    rq, rrem = divmod(nrch, _NS)

    @functools.partial(
        pl.kernel,
        out_type=jax.ShapeDtypeStruct((_NC, N, D), jnp.float32),
        mesh=_mesh(),
        scratch_types=[
            pltpu.VMEM((CH,), jnp.int32),
            pltpu.VMEM((CH, D), jnp.float32),
            pltpu.VMEM((RB, D), jnp.float32),
            pltpu.VMEM((RB, D), jnp.float32),
            pltpu.VMEM_SHARED((N, D), jnp.float32),
            pltpu.SemaphoreType.DMA,
        ],
    )
    def k(vals_ref, dst_ref, out_ref, idx_v, rows_v, zbuf, obuf, acc, sem):
        cid = lax.axis_index("c")
        sid = lax.axis_index("s")
        wid = sid * _NC + cid
        rcnt = rq + jnp.where(sid < rrem, 1, 0)

        for r in range(RB):
            for c in range(_LANES):
                zbuf[r, pl.ds(c * 16, 16)] = jnp.zeros((16,), jnp.float32)

        @pl.loop(0, rcnt)
        def _(j):
            pltpu.sync_copy(zbuf, acc.at[pl.ds((sid + j * _NS) * RB, RB)])

        plsc.subcore_barrier()

        cnt = q + jnp.where(wid < rem, 1, 0)

        @pl.loop(0, cnt)
        def _(i):
            ch = wid + i * _NW
            pltpu.sync_copy(dst_ref.at[pl.ds(ch * CH, CH)], idx_v)
            pltpu.sync_copy(vals_ref.at[pl.ds(ch * CH, CH)], rows_v)
            pltpu.sync_copy(rows_v, acc.at[idx_v], add=True)

        plsc.subcore_barrier()

        @pl.loop(0, rcnt)
        def _(j):
            off = (sid + j * _NS) * RB
            pltpu.sync_copy(acc.at[pl.ds(off, RB)], obuf)
            pltpu.sync_copy(obuf, out_ref.at[cid, pl.ds(off, RB)])

    return k(vals, dst)


def _relu_half(h):
    col = lax.broadcasted_iota(jnp.int32, h.shape, 1)
    return jnp.where(col >= D // 2, jnp.maximum(h, 0.0), h)


def _pre_body(refs, h_ref, stats_ref):
    (base_ref, deg_ref, sa_ref, sb_ref, o1_ref, o2_ref,
     wm, wd, wa, wb, wo, bias_ref) = refs
    b = base_ref[...]
    h = jnp.dot(b, wm[...], preferred_element_type=jnp.float32)
    h = h + jnp.dot(b * deg_ref[...], wd[...], preferred_element_type=jnp.float32)
    h = h + jnp.dot(sa_ref[...], wa[...], preferred_element_type=jnp.float32)
    h = h + jnp.dot(sb_ref[...], wb[...], preferred_element_type=jnp.float32)
    other = o1_ref[...] if o2_ref is None else o1_ref[...] + o2_ref[...]
    h = h + jnp.dot(other, wo[...], preferred_element_type=jnp.float32)
    h = h + bias_ref[...]
    h = _relu_half(h)
    h_ref[...] = h

    @pl.when(pl.program_id(0) == 0)
    def _():
        stats_ref[...] = jnp.zeros_like(stats_ref)

    stats_ref[0:1, :] = stats_ref[0:1, :] + jnp.sum(h, axis=0, keepdims=True)
    stats_ref[1:2, :] = stats_ref[1:2, :] + jnp.sum(h * h, axis=0, keepdims=True)


def _tc_pre(base, deg, sa, sb, others, wm, wd, wa, wb, wo, bias, *, blk):
    rows = base.shape[0]
    grid = rows // blk
    two = len(others) == 2

    def body(base_ref, deg_ref, sa_ref, sb_ref, o1_ref, *rest):
        if two:
            o2_ref, wm_r, wd_r, wa_r, wb_r, wo_r, bias_ref, h_ref, stats_ref = rest
        else:
            wm_r, wd_r, wa_r, wb_r, wo_r, bias_ref, h_ref, stats_ref = rest
            o2_ref = None
        _pre_body((base_ref, deg_ref, sa_ref, sb_ref, o1_ref, o2_ref,
                   wm_r, wd_r, wa_r, wb_r, wo_r, bias_ref), h_ref, stats_ref)

    row_spec = pl.BlockSpec((blk, D), lambda i: (i, 0))
    full_spec = pl.BlockSpec((D, D), lambda i: (0, 0))
    vec_spec = pl.BlockSpec((1, D), lambda i: (0, 0))
    in_specs = [row_spec, pl.BlockSpec((blk, 1), lambda i: (i, 0)),
                row_spec, row_spec, row_spec]
    if two:
        in_specs.append(row_spec)
    in_specs += [full_spec] * 5 + [vec_spec]

    return pl.pallas_call(
        body,
        grid=(grid,),
        in_specs=in_specs,
        out_specs=[row_spec, pl.BlockSpec((8, D), lambda i: (0, 0))],
        out_shape=[jax.ShapeDtypeStruct((rows, D), jnp.float32),
                   jax.ShapeDtypeStruct((8, D), jnp.float32)],
    )(base, deg, sa, sb, *others, wm, wd, wa, wb, wo, bias)


def _tc_bn(h, stats, s, b, *, blk):
    rows = h.shape[0]
    grid = rows // blk
    inv_n = 1.0 / rows

    def body(h_ref, stats_ref, s_ref, b_ref, o_ref):
        m = stats_ref[0:1, :] * inv_n
        v = stats_ref[1:2, :] * inv_n - m * m
        scale = lax.rsqrt(v + 1e-5) * s_ref[...]
        o_ref[...] = (h_ref[...] - m) * scale + b_ref[...]

    row_spec = pl.BlockSpec((blk, D), lambda i: (i, 0))
    vec_spec = pl.BlockSpec((1, D), lambda i: (0, 0))
    return pl.pallas_call(
        body,
        grid=(grid,),
        in_specs=[row_spec, pl.BlockSpec((8, D), lambda i: (0, 0)),
                  vec_spec, vec_spec],
        out_specs=row_spec,
        out_shape=jax.ShapeDtypeStruct((rows, D), jnp.float32),
    )(h, stats, s.reshape(1, D), b.reshape(1, D))


def kernel(x, y, deg_g, deg_lg, pm_pd, g_t, g_tt, lg_t, lg_tt, edge_dst,
           W_tx, b_tx, W_td, b_td, W_ty, b_ty, W_t0, b_t0, W_t1, b_t1,
           W_gy, b_gy, W_gd, b_gd, W_gx, b_gx, W_g0, b_g0, W_g1, b_g1,
           bnx_s, bnx_b, bny_s, bny_b):
    sxt, sxtt = _sc_gather_fused(x, g_t.reshape(-1), g_tt.reshape(-1),
                                 x, pm_pd, rows=N, with_p=False)
    syt, sytt, px = _sc_gather_fused(y, lg_t.reshape(-1), lg_tt.reshape(-1),
                                     x, pm_pd, rows=E, with_p=True)
    py = _sc_scatter_add(y, edge_dst)

    bias_x = (b_tx + b_td + b_t0 + b_t1 + b_ty).reshape(1, D)
    bias_y = (b_gy + b_gd + b_g0 + b_g1 + b_gx).reshape(1, D)

    hx, stx = _tc_pre(x, deg_g, sxt, sxtt, [py[0], py[1]],
                      W_tx.T, W_td.T, W_t0.T, W_t1.T, W_ty.T, bias_x,
                      blk=2000)
    hy, sty = _tc_pre(y, deg_lg, syt, sytt, [px],
                      W_gy.T, W_gd.T, W_g0.T, W_g1.T, W_gx.T, bias_y,
                      blk=2000)

    xn = _tc_bn(hx, stx, bnx_s, bnx_b, blk=2000)
    yn = _tc_bn(hy, sty, bny_s, bny_b, blk=2000)
    return (xn, yn)
